# Initial kernel scaffold; baseline (speedup 1.0000x reference)
import jax
import jax.numpy as jnp
from jax.experimental import pallas as pl


def kernel(x, edge_index, W1, b1, gamma1, beta1, W2, b2, gamma2, beta2, W3, b3):
    raise NotImplementedError("write your pallas kernel here")


# speedup vs baseline: 11.0722x; 11.0722x over previous
import functools

import jax
import jax.numpy as jnp
from jax import lax
from jax.experimental import pallas as pl
from jax.experimental.pallas import tpu as pltpu
from jax.experimental.pallas import tpu_sc as plsc

NC = 2
NS = 16
FQ = 64
EPS = 1e-5
F32 = jnp.float32


def _sc_mesh():
  return plsc.VectorSubcoreMesh(
      core_axis_name="c", subcore_axis_name="s",
      num_cores=NC, num_subcores=NS)


_SC_PARAMS = pltpu.CompilerParams(use_tc_tiling_on_sc=False)


def _make_deg_kernel(NP, ROWS):
  RT = ROWS // (NC * NS)
  RN = NP // NS

  @functools.partial(
      pl.kernel,
      out_type=jax.ShapeDtypeStruct((2 * NP,), F32),
      mesh=_sc_mesh(),
      compiler_params=_SC_PARAMS,
      scratch_types=[
          pltpu.VMEM((RT, 128), jnp.int32),
          pltpu.VMEM((128,), F32),
          pltpu.VMEM_SHARED((NP,), F32),
          pltpu.SemaphoreType.DMA,
      ],
  )
  def deg_kernel(dst2d, ones_hbm, zeros_hbm, out, didx, ones_v, acc, sem):
    del sem
    c = lax.axis_index("c")
    s = lax.axis_index("s")
    w = c * NS + s
    pltpu.sync_copy(dst2d.at[pl.ds(w * RT, RT)], didx)
    pltpu.sync_copy(ones_hbm, ones_v)
    pltpu.sync_copy(zeros_hbm.at[pl.ds(s * RN, RN)], acc.at[pl.ds(s * RN, RN)])
    plsc.subcore_barrier()

    def step(j, carry):
      pltpu.sync_copy(ones_v, acc.at[didx.at[j]], add=True)
      return carry

    lax.fori_loop(0, RT, step, 0)
    plsc.subcore_barrier()
    pltpu.sync_copy(acc.at[pl.ds(s * RN, RN)],
                    out.at[pl.ds(c * NP + s * RN, RN)])

  return deg_kernel


def _make_agg_kernel(NP, ROWS):
  RT = ROWS // NS
  RN = NP // NS

  @functools.partial(
      pl.kernel,
      out_type=[jax.ShapeDtypeStruct((NP, FQ), F32),
                jax.ShapeDtypeStruct((NP, FQ), F32)],
      mesh=_sc_mesh(),
      compiler_params=_SC_PARAMS,
      scratch_types=[
          pltpu.VMEM((RT, 128), jnp.int32),
          pltpu.VMEM((RT, 128), jnp.int32),
          pltpu.VMEM((128, FQ), F32),
          pltpu.VMEM((128, FQ), F32),
          pltpu.VMEM_SHARED((NP, FQ), F32),
          pltpu.SemaphoreType.DMA,
          pltpu.SemaphoreType.DMA,
      ],
  )
  def agg_kernel(ya, yb, src2d, dst2d, oa, ob,
                 sidx, didx, r0, r1, acc, sem0, sem1):
    c = lax.axis_index("c")
    s = lax.axis_index("s")
    pltpu.sync_copy(src2d.at[pl.ds(s * RT, RT)], sidx)
    pltpu.sync_copy(dst2d.at[pl.ds(s * RT, RT)], didx)

    def run(y_ref, out_ref):
      pltpu.sync_copy(y_ref.at[pl.ds(s * RN, RN)], acc.at[pl.ds(s * RN, RN)])
      plsc.subcore_barrier()

      pltpu.async_copy(y_ref.at[sidx.at[0]], r0, sem0)

      def step(i, carry):
        jj = 2 * i
        pltpu.make_async_copy(y_ref.at[sidx.at[jj]], r0, sem0).wait()
        pltpu.async_copy(y_ref.at[sidx.at[jj + 1]], r1, sem1)
        pltpu.sync_copy(r0, acc.at[didx.at[jj]], add=True)
        pltpu.make_async_copy(y_ref.at[sidx.at[jj + 1]], r1, sem1).wait()

        @pl.when(jj + 2 < RT)
        def _():
          pltpu.async_copy(y_ref.at[sidx.at[jj + 2]], r0, sem0)

        pltpu.sync_copy(r1, acc.at[didx.at[jj + 1]], add=True)
        return carry

      lax.fori_loop(0, RT // 2, step, 0)
      plsc.subcore_barrier()
      pltpu.sync_copy(acc.at[pl.ds(s * RN, RN)], out_ref.at[pl.ds(s * RN, RN)])

    pl.when(c == 0)(lambda: run(ya, oa))
    pl.when(c == 1)(lambda: run(yb, ob))

  return agg_kernel


def _quarter_specs(BM, n, width=FQ):
  return [pl.BlockSpec((BM, width), lambda m: (m, 0)) for _ in range(n)]


def _make_first_matmul(NP, F, H, BM):
  NQ = H // FQ

  def body(x_ref, h0_ref, h1_ref, w_ref, *o_refs):
    dis = lax.rsqrt(h0_ref[...] + h1_ref[...] + 1.0)
    o_refs[NQ][...] = dis
    y = jnp.dot(x_ref[...] * dis, w_ref[...], preferred_element_type=F32)
    for q in range(NQ):
      o_refs[q][...] = y[:, q * FQ:(q + 1) * FQ]

  return pl.pallas_call(
      body,
      grid=(NP // BM,),
      in_specs=[
          pl.BlockSpec((BM, F), lambda m: (m, 0)),
          pl.BlockSpec((BM, 1), lambda m: (m, 0)),
          pl.BlockSpec((BM, 1), lambda m: (m, 0)),
          pl.BlockSpec((F, H), lambda m: (0, 0)),
      ],
      out_specs=_quarter_specs(BM, NQ) + [pl.BlockSpec((BM, 1), lambda m: (m, 0))],
      out_shape=[jax.ShapeDtypeStruct((NP, FQ), F32) for _ in range(NQ)]
      + [jax.ShapeDtypeStruct((NP, 1), F32)],
  )


def _make_stats(NP, F, BM, n_real):
  NQ = F // FQ

  def body(*refs):
    a_refs, dis_ref, o_ref = refs[:NQ], refs[NQ], refs[NQ + 1]
    m = pl.program_id(0)
    w = jnp.concatenate([r[...] for r in a_refs], axis=1) * dis_ref[...]
    rows = lax.broadcasted_iota(jnp.int32, (BM, 1), 0) + m * BM
    w = jnp.where(rows < n_real, w, 0.0)

    @pl.when(m == 0)
    def _():
      o_ref[...] = jnp.zeros_like(o_ref)

    o_ref[0:1, :] += jnp.sum(w, axis=0, keepdims=True)
    o_ref[1:2, :] += jnp.sum(w * w, axis=0, keepdims=True)

  return pl.pallas_call(
      body,
      grid=(NP // BM,),
      in_specs=_quarter_specs(BM, NQ) + [pl.BlockSpec((BM, 1), lambda m: (m, 0))],
      out_specs=pl.BlockSpec((8, F), lambda m: (0, 0)),
      out_shape=jax.ShapeDtypeStruct((8, F), F32),
  )


def _make_bn_matmul(NP, F, H, BM, n_real):
  NQI = F // FQ
  NQO = H // FQ

  def body(*refs):
    a_refs = refs[:NQI]
    dis_ref, st_ref, g_ref, be_ref, w_ref = refs[NQI:NQI + 5]
    o_refs = refs[NQI + 5:]
    dis = dis_ref[...]
    wv = jnp.concatenate([r[...] for r in a_refs], axis=1) * dis
    mean = st_ref[0:1, :] * (1.0 / n_real)
    var = st_ref[1:2, :] * (1.0 / n_real) - mean * mean
    inv = lax.rsqrt(var + EPS)
    h = jnp.maximum((wv - mean) * inv * g_ref[...] + be_ref[...], 0.0)
    y = jnp.dot(h * dis, w_ref[...], preferred_element_type=F32)
    for q in range(NQO):
      o_refs[q][...] = y[:, q * FQ:(q + 1) * FQ]

  return pl.pallas_call(
      body,
      grid=(NP // BM,),
      in_specs=_quarter_specs(BM, NQI) + [
          pl.BlockSpec((BM, 1), lambda m: (m, 0)),
          pl.BlockSpec((8, F), lambda m: (0, 0)),
          pl.BlockSpec((1, F), lambda m: (0, 0)),
          pl.BlockSpec((1, F), lambda m: (0, 0)),
          pl.BlockSpec((F, H), lambda m: (0, 0)),
      ],
      out_specs=_quarter_specs(BM, NQO),
      out_shape=[jax.ShapeDtypeStruct((NP, FQ), F32) for _ in range(NQO)],
  )


def _make_final(NP, C, BM):
  NQ = C // FQ

  def body(*refs):
    a_refs, dis_ref, b_ref, o_ref = refs[:NQ], refs[NQ], refs[NQ + 1], refs[NQ + 2]
    o_ref[...] = (jnp.concatenate([r[...] for r in a_refs], axis=1)
                  * dis_ref[...] + b_ref[...])

  return pl.pallas_call(
      body,
      grid=(NP // BM,),
      in_specs=_quarter_specs(BM, NQ) + [
          pl.BlockSpec((BM, 1), lambda m: (m, 0)),
          pl.BlockSpec((1, C), lambda m: (0, 0)),
      ],
      out_specs=pl.BlockSpec((BM, C), lambda m: (m, 0)),
      out_shape=jax.ShapeDtypeStruct((NP, C), F32),
  )


def kernel(x, edge_index, W1, b1, gamma1, beta1, W2, b2, gamma2, beta2,
           W3, b3):
  del b1, b2
  N, F = x.shape
  E = edge_index.shape[1]
  H = W1.shape[1]
  C = W3.shape[1]

  NP = -(-(N + 16) // 1024) * 1024
  rows_e = -(-E // 128)
  ROWS = -(-rows_e // 32) * 32
  PAD_E = ROWS * 128
  BM = 1024
  dummy0 = NP - 16

  x_p = jnp.zeros((NP, F), F32).at[:N, :].set(x)
  ids = jnp.arange(PAD_E - E, dtype=jnp.int32)
  src2d = jnp.concatenate([edge_index[0], ids % N]).reshape(ROWS, 128)
  dst2d = jnp.concatenate([edge_index[1], dummy0 + (ids % 16)]
                          ).reshape(ROWS, 128)
  ones_v = jnp.ones((128,), F32)
  zeros_v = jnp.zeros((NP,), F32)
  g1 = gamma1.reshape(1, H)
  be1 = beta1.reshape(1, H)
  g2 = gamma2.reshape(1, H)
  be2 = beta2.reshape(1, H)
  b3r = b3.reshape(1, C)

  agg = _make_agg_kernel(NP, ROWS)

  hist = _make_deg_kernel(NP, ROWS)(dst2d, ones_v, zeros_v)
  h0 = hist[:NP].reshape(NP, 1)
  h1 = hist[NP:].reshape(NP, 1)

  y0, y1, y2, y3, dis = _make_first_matmul(NP, F, H, BM)(x_p, h0, h1, W1)
  a0, a1 = agg(y0, y1, src2d, dst2d)
  a2, a3 = agg(y2, y3, src2d, dst2d)
  st1 = _make_stats(NP, H, BM, N)(a0, a1, a2, a3, dis)

  y0, y1, y2, y3 = _make_bn_matmul(NP, H, H, BM, N)(
      a0, a1, a2, a3, dis, st1, g1, be1, W2)
  a0, a1 = agg(y0, y1, src2d, dst2d)
  a2, a3 = agg(y2, y3, src2d, dst2d)
  st2 = _make_stats(NP, H, BM, N)(a0, a1, a2, a3, dis)

  z0, z1 = _make_bn_matmul(NP, H, C, BM, N)(
      a0, a1, a2, a3, dis, st2, g2, be2, W3)
  c0, c1 = agg(z0, z1, src2d, dst2d)
  out = _make_final(NP, C, BM)(c0, c1, dis, b3r)

  return out[:N, :]


# speedup vs baseline: 14.7241x; 1.3298x over previous
import functools

import jax
import jax.numpy as jnp
from jax import lax
from jax.experimental import pallas as pl
from jax.experimental.pallas import tpu as pltpu
from jax.experimental.pallas import tpu_sc as plsc

NC = 2
NS = 16
FQ = 64
EPS = 1e-5
F32 = jnp.float32


def _sc_mesh():
  return plsc.VectorSubcoreMesh(
      core_axis_name="c", subcore_axis_name="s",
      num_cores=NC, num_subcores=NS)


_SC_PARAMS = pltpu.CompilerParams(use_tc_tiling_on_sc=False)


def _make_deg_kernel(NP, ROWS):
  RT = ROWS // (NC * NS)
  RN = NP // NS

  @functools.partial(
      pl.kernel,
      out_type=jax.ShapeDtypeStruct((2 * NP,), F32),
      mesh=_sc_mesh(),
      compiler_params=_SC_PARAMS,
      scratch_types=[
          pltpu.VMEM((RT, 128), jnp.int32),
          pltpu.VMEM((128,), F32),
          pltpu.VMEM_SHARED((NP,), F32),
          pltpu.SemaphoreType.DMA,
      ],
  )
  def deg_kernel(dst2d, ones_hbm, zeros_hbm, out, didx, ones_v, acc, sem):
    del sem
    c = lax.axis_index("c")
    s = lax.axis_index("s")
    w = c * NS + s
    pltpu.sync_copy(dst2d.at[pl.ds(w * RT, RT)], didx)
    pltpu.sync_copy(ones_hbm, ones_v)
    pltpu.sync_copy(zeros_hbm.at[pl.ds(s * RN, RN)], acc.at[pl.ds(s * RN, RN)])
    plsc.subcore_barrier()

    def step(j, carry):
      pltpu.sync_copy(ones_v, acc.at[didx.at[j]], add=True)
      return carry

    lax.fori_loop(0, RT, step, 0)
    plsc.subcore_barrier()
    pltpu.sync_copy(acc.at[pl.ds(s * RN, RN)],
                    out.at[pl.ds(c * NP + s * RN, RN)])

  return deg_kernel


def _make_agg_kernel(NP, ROWS):
  RT = ROWS // NS
  RN = NP // NS
  NB = 8
  LA = 4
  assert RT % NB == 0 and RT >= 2 * NB

  @functools.partial(
      pl.kernel,
      out_type=[jax.ShapeDtypeStruct((NP, FQ), F32),
                jax.ShapeDtypeStruct((NP, FQ), F32)],
      mesh=_sc_mesh(),
      compiler_params=_SC_PARAMS,
      scratch_types=[
          pltpu.VMEM((RT, 128), jnp.int32),
          pltpu.VMEM((RT, 128), jnp.int32),
          [pltpu.VMEM((128, FQ), F32) for _ in range(NB)],
          pltpu.VMEM_SHARED((NP, FQ), F32),
          [pltpu.SemaphoreType.DMA for _ in range(NB)],
          [pltpu.SemaphoreType.DMA for _ in range(NB)],
      ],
  )
  def agg_kernel(ya, yb, src2d, dst2d, oa, ob,
                 sidx, didx, rbufs, acc, gsems, ssems):
    c = lax.axis_index("c")
    s = lax.axis_index("s")
    pltpu.sync_copy(src2d.at[pl.ds(s * RT, RT)], sidx)
    pltpu.sync_copy(dst2d.at[pl.ds(s * RT, RT)], didx)

    def run(y_ref, out_ref):
      pltpu.sync_copy(y_ref.at[pl.ds(s * RN, RN)], acc.at[pl.ds(s * RN, RN)])
      plsc.subcore_barrier()

      for b in range(LA):
        pltpu.async_copy(y_ref.at[sidx.at[b]], rbufs[b], gsems[b])

      def outer(i, carry):
        for b in range(NB):
          k = NB * i + b
          pltpu.make_async_copy(y_ref.at[sidx.at[k]], rbufs[b], gsems[b]).wait()
          pltpu.async_copy(rbufs[b], acc.at[didx.at[k]], ssems[b], add=True)
          bb = (b + LA) % NB

          @pl.when(k + LA < RT)
          def _():
            @pl.when(k + LA >= NB)
            def _():
              pltpu.make_async_copy(
                  rbufs[bb], acc.at[didx.at[k + LA - NB]], ssems[bb]).wait()

            pltpu.async_copy(y_ref.at[sidx.at[k + LA]], rbufs[bb], gsems[bb])

        return carry

      lax.fori_loop(0, RT // NB, outer, 0)
      for t in range(LA):
        k = RT - LA + t
        b = k % NB
        pltpu.make_async_copy(rbufs[b], acc.at[didx.at[k]], ssems[b]).wait()
      plsc.subcore_barrier()
      pltpu.sync_copy(acc.at[pl.ds(s * RN, RN)], out_ref.at[pl.ds(s * RN, RN)])

    pl.when(c == 0)(lambda: run(ya, oa))
    pl.when(c == 1)(lambda: run(yb, ob))

  return agg_kernel


def _quarter_specs(BM, n, width=FQ):
  return [pl.BlockSpec((BM, width), lambda m: (m, 0)) for _ in range(n)]


def _make_first_matmul(NP, F, H, BM):
  NQ = H // FQ

  def body(x_ref, h0_ref, h1_ref, w_ref, *o_refs):
    dis = lax.rsqrt(h0_ref[...] + h1_ref[...] + 1.0)
    o_refs[NQ][...] = dis
    y = jnp.dot(x_ref[...] * dis, w_ref[...], preferred_element_type=F32)
    for q in range(NQ):
      o_refs[q][...] = y[:, q * FQ:(q + 1) * FQ]

  return pl.pallas_call(
      body,
      grid=(NP // BM,),
      in_specs=[
          pl.BlockSpec((BM, F), lambda m: (m, 0)),
          pl.BlockSpec((BM, 1), lambda m: (m, 0)),
          pl.BlockSpec((BM, 1), lambda m: (m, 0)),
          pl.BlockSpec((F, H), lambda m: (0, 0)),
      ],
      out_specs=_quarter_specs(BM, NQ) + [pl.BlockSpec((BM, 1), lambda m: (m, 0))],
      out_shape=[jax.ShapeDtypeStruct((NP, FQ), F32) for _ in range(NQ)]
      + [jax.ShapeDtypeStruct((NP, 1), F32)],
  )


def _make_stats(NP, F, BM, n_real):
  NQ = F // FQ

  def body(*refs):
    a_refs, dis_ref, o_ref = refs[:NQ], refs[NQ], refs[NQ + 1]
    m = pl.program_id(0)
    w = jnp.concatenate([r[...] for r in a_refs], axis=1) * dis_ref[...]
    rows = lax.broadcasted_iota(jnp.int32, (BM, 1), 0) + m * BM
    w = jnp.where(rows < n_real, w, 0.0)

    @pl.when(m == 0)
    def _():
      o_ref[...] = jnp.zeros_like(o_ref)

    o_ref[0:1, :] += jnp.sum(w, axis=0, keepdims=True)
    o_ref[1:2, :] += jnp.sum(w * w, axis=0, keepdims=True)

  return pl.pallas_call(
      body,
      grid=(NP // BM,),
      in_specs=_quarter_specs(BM, NQ) + [pl.BlockSpec((BM, 1), lambda m: (m, 0))],
      out_specs=pl.BlockSpec((8, F), lambda m: (0, 0)),
      out_shape=jax.ShapeDtypeStruct((8, F), F32),
  )


def _make_bn_matmul(NP, F, H, BM, n_real):
  NQI = F // FQ
  NQO = H // FQ

  def body(*refs):
    a_refs = refs[:NQI]
    dis_ref, st_ref, g_ref, be_ref, w_ref = refs[NQI:NQI + 5]
    o_refs = refs[NQI + 5:]
    dis = dis_ref[...]
    wv = jnp.concatenate([r[...] for r in a_refs], axis=1) * dis
    mean = st_ref[0:1, :] * (1.0 / n_real)
    var = st_ref[1:2, :] * (1.0 / n_real) - mean * mean
    inv = lax.rsqrt(var + EPS)
    h = jnp.maximum((wv - mean) * inv * g_ref[...] + be_ref[...], 0.0)
    y = jnp.dot(h * dis, w_ref[...], preferred_element_type=F32)
    for q in range(NQO):
      o_refs[q][...] = y[:, q * FQ:(q + 1) * FQ]

  return pl.pallas_call(
      body,
      grid=(NP // BM,),
      in_specs=_quarter_specs(BM, NQI) + [
          pl.BlockSpec((BM, 1), lambda m: (m, 0)),
          pl.BlockSpec((8, F), lambda m: (0, 0)),
          pl.BlockSpec((1, F), lambda m: (0, 0)),
          pl.BlockSpec((1, F), lambda m: (0, 0)),
          pl.BlockSpec((F, H), lambda m: (0, 0)),
      ],
      out_specs=_quarter_specs(BM, NQO),
      out_shape=[jax.ShapeDtypeStruct((NP, FQ), F32) for _ in range(NQO)],
  )


def _make_final(NP, C, BM):
  NQ = C // FQ

  def body(*refs):
    a_refs, dis_ref, b_ref, o_ref = refs[:NQ], refs[NQ], refs[NQ + 1], refs[NQ + 2]
    o_ref[...] = (jnp.concatenate([r[...] for r in a_refs], axis=1)
                  * dis_ref[...] + b_ref[...])

  return pl.pallas_call(
      body,
      grid=(NP // BM,),
      in_specs=_quarter_specs(BM, NQ) + [
          pl.BlockSpec((BM, 1), lambda m: (m, 0)),
          pl.BlockSpec((1, C), lambda m: (0, 0)),
      ],
      out_specs=pl.BlockSpec((BM, C), lambda m: (m, 0)),
      out_shape=jax.ShapeDtypeStruct((NP, C), F32),
  )


def kernel(x, edge_index, W1, b1, gamma1, beta1, W2, b2, gamma2, beta2,
           W3, b3):
  del b1, b2
  N, F = x.shape
  E = edge_index.shape[1]
  H = W1.shape[1]
  C = W3.shape[1]

  NP = -(-(N + 16) // 1024) * 1024
  rows_e = -(-E // 128)
  ROWS = -(-rows_e // 32) * 32
  PAD_E = ROWS * 128
  BM = 1024
  dummy0 = NP - 16

  x_p = jnp.zeros((NP, F), F32).at[:N, :].set(x)
  ids = jnp.arange(PAD_E - E, dtype=jnp.int32)
  src2d = jnp.concatenate([edge_index[0], ids % N]).reshape(ROWS, 128)
  dst2d = jnp.concatenate([edge_index[1], dummy0 + (ids % 16)]
                          ).reshape(ROWS, 128)
  ones_v = jnp.ones((128,), F32)
  zeros_v = jnp.zeros((NP,), F32)
  g1 = gamma1.reshape(1, H)
  be1 = beta1.reshape(1, H)
  g2 = gamma2.reshape(1, H)
  be2 = beta2.reshape(1, H)
  b3r = b3.reshape(1, C)

  agg = _make_agg_kernel(NP, ROWS)

  hist = _make_deg_kernel(NP, ROWS)(dst2d, ones_v, zeros_v)
  h0 = hist[:NP].reshape(NP, 1)
  h1 = hist[NP:].reshape(NP, 1)

  y0, y1, y2, y3, dis = _make_first_matmul(NP, F, H, BM)(x_p, h0, h1, W1)
  a0, a1 = agg(y0, y1, src2d, dst2d)
  a2, a3 = agg(y2, y3, src2d, dst2d)
  st1 = _make_stats(NP, H, BM, N)(a0, a1, a2, a3, dis)

  y0, y1, y2, y3 = _make_bn_matmul(NP, H, H, BM, N)(
      a0, a1, a2, a3, dis, st1, g1, be1, W2)
  a0, a1 = agg(y0, y1, src2d, dst2d)
  a2, a3 = agg(y2, y3, src2d, dst2d)
  st2 = _make_stats(NP, H, BM, N)(a0, a1, a2, a3, dis)

  z0, z1 = _make_bn_matmul(NP, H, C, BM, N)(
      a0, a1, a2, a3, dis, st2, g2, be2, W3)
  c0, c1 = agg(z0, z1, src2d, dst2d)
  out = _make_final(NP, C, BM)(c0, c1, dis, b3r)

  return out[:N, :]
